# Initial kernel scaffold; baseline (speedup 1.0000x reference)
import jax
import jax.numpy as jnp
from jax.experimental import pallas as pl


def kernel(x, neighbor, W1, b1, gamma1, beta1, W2, b2, gamma2, beta2, Wc, bc):
    raise NotImplementedError("write your pallas kernel here")


# speedup vs baseline: 1.0303x; 1.0303x over previous
import jax
import jax.numpy as jnp
from jax.experimental import pallas as pl
_NB = 40


def _sgnroot(v):
    return jnp.sign(v) * jnp.sqrt(jnp.abs(v))


def _colnorm(t):
    den = jnp.sum(jnp.abs(t), axis=1, keepdims=True) + 1e-7
    y = t / den
    return jnp.where(y != y, 0.0, y)


def _ktrans_block(x_ref, nbr_ref, w1t_ref, g1v_ref, b1v_ref, w2t_ref,
                  gb2g_ref, gb2b_ref, b2_ref, wct_ref, bc_ref, out_ref):
    xb = x_ref[...]
    nb = nbr_ref[...]
    NB, D, F = nb.shape

    s = jnp.sum(nb, axis=1)
    fadj = xb[:, :, None] * s[:, None, :] + s[:, :, None] * xb[:, None, :]
    adj = _colnorm(_sgnroot(fadj))

    w1t = w1t_ref[...]
    a1 = jax.lax.dot_general(
        adj.reshape(NB * F, F), w1t, (((1,), (0,)), ((), ())),
        preferred_element_type=jnp.float32).reshape(NB, F, 64)

    x1 = jnp.sum(xb[:, :, None] * a1, axis=1)
    n2 = jax.lax.dot_general(
        nb, a1, (((2,), (1,)), ((0,), (0,))),
        preferred_element_type=jnp.float32)

    inv = 1.0 / jnp.sqrt(1.0 + 1e-5)
    g1v = g1v_ref[...] * inv
    b1v = b1v_ref[...]
    x1 = x1 * g1v + b1v
    x1 = x1 / (1.0 + jnp.abs(x1))
    n2 = n2 * g1v[None] + b1v[None]
    n2 = n2 / (1.0 + jnp.abs(n2))

    x1r = x1.reshape(NB, 4, 16)
    n2s = jnp.sum(n2, axis=1).reshape(NB, 4, 16)
    fadj2 = jnp.sum(
        x1r[:, :, :, None] * n2s[:, :, None, :]
        + n2s[:, :, :, None] * x1r[:, :, None, :], axis=1)
    adj2 = _colnorm(_sgnroot(fadj2))

    xt2 = jnp.sum(x1r[:, :, :, None] * adj2[:, None, :, :], axis=2)
    x2 = jax.lax.dot_general(
        xt2.reshape(NB, 64), w2t_ref[...], (((1,), (0,)), ((), ())),
        preferred_element_type=jnp.float32) + b2_ref[...]
    x2 = x2 * (gb2g_ref[...] * inv) + gb2b_ref[...]
    x2 = x2 / (1.0 + jnp.abs(x2))

    out_ref[...] = jax.lax.dot_general(
        x2, wct_ref[...], (((1,), (0,)), ((), ())),
        preferred_element_type=jnp.float32) + bc_ref[...]


@jax.jit
def kernel(x, neighbor, W1, b1, gamma1, beta1, W2, b2, gamma2, beta2, Wc, bc):
    N, _, F = x.shape
    D = neighbor.shape[2]
    xb = x.reshape(N, F)
    nb = neighbor.reshape(N, D, F)

    w1t = W1.reshape(64, F).T
    g1v = jnp.repeat(gamma1, 16)[None, :]
    inv = 1.0 / jnp.sqrt(1.0 + 1e-5)
    b1v = (b1 * inv * g1v[0] + jnp.repeat(beta1, 16))[None, :]

    w2t = W2.reshape(32, 64).T
    gb2g = gamma2[None, :]
    gb2b = beta2[None, :]
    b2r = b2[None, :]
    wct = Wc.T
    bcr = bc[None, :]

    grid = (N // _NB,)
    out = pl.pallas_call(
        _ktrans_block,
        grid=grid,
        in_specs=[
            pl.BlockSpec((_NB, F), lambda i: (i, 0)),
            pl.BlockSpec((_NB, D, F), lambda i: (i, 0, 0)),
            pl.BlockSpec((F, 64), lambda i: (0, 0)),
            pl.BlockSpec((1, 64), lambda i: (0, 0)),
            pl.BlockSpec((1, 64), lambda i: (0, 0)),
            pl.BlockSpec((64, 32), lambda i: (0, 0)),
            pl.BlockSpec((1, 32), lambda i: (0, 0)),
            pl.BlockSpec((1, 32), lambda i: (0, 0)),
            pl.BlockSpec((1, 32), lambda i: (0, 0)),
            pl.BlockSpec((32, 10), lambda i: (0, 0)),
            pl.BlockSpec((1, 10), lambda i: (0, 0)),
        ],
        out_specs=pl.BlockSpec((_NB, 10), lambda i: (i, 0)),
        out_shape=jax.ShapeDtypeStruct((N, 10), jnp.float32),
    )(xb, nb, w1t, g1v, b1v, w2t, gb2g, gb2b, b2r, wct, bcr)
    return out


# speedup vs baseline: 3.1710x; 3.0776x over previous
import jax
import jax.numpy as jnp
from jax.experimental import pallas as pl

_NB = 128


def _sgnroot(v):
    return jnp.sign(v) * jnp.sqrt(jnp.abs(v))


def _colnorm(t, axis):
    recip = 1.0 / (jnp.sum(jnp.abs(t), axis=axis, keepdims=True) + 1e-7)
    y = t * recip
    return jnp.where(y != y, 0.0, y)


def _ktrans_block(x_ref, nbr_ref, w1t_ref, g1v_ref, b1v_ref, w2f_ref,
                  g2_ref, bb2_ref, wc_ref, bc_ref, out_ref):
    xb = x_ref[...]
    nb = nbr_ref[...]
    NB, D, F = nb.shape

    s = jnp.sum(nb, axis=1)
    fadj = xb[:, :, None] * s[:, None, :] + s[:, :, None] * xb[:, None, :]
    adj = _colnorm(_sgnroot(fadj), axis=1)

    a1 = jax.lax.dot_general(
        adj.reshape(NB * F, F), w1t_ref[...], (((1,), (0,)), ((), ())),
        preferred_element_type=jnp.float32).reshape(NB, F, 64)

    x1 = jnp.sum(xb[:, :, None] * a1, axis=1)
    n2 = jax.lax.dot_general(
        nb, a1, (((2,), (1,)), ((0,), (0,))),
        preferred_element_type=jnp.float32)

    g1v = g1v_ref[...]
    b1v = b1v_ref[...]
    x1 = x1 * g1v + b1v
    x1 = x1 / (1.0 + jnp.abs(x1))
    n2 = n2 * g1v[None] + b1v[None]
    n2 = n2 / (1.0 + jnp.abs(n2))
    n2s = jnp.sum(n2, axis=1)

    x1t = x1.T.reshape(4, 16, NB)
    n2t = n2s.T.reshape(4, 16, NB)
    fadj2 = jnp.sum(
        x1t[:, :, None, :] * n2t[:, None, :, :]
        + n2t[:, :, None, :] * x1t[:, None, :, :], axis=0)
    adj2 = _colnorm(_sgnroot(fadj2), axis=0)

    xt2 = jnp.sum(x1t[:, :, None, :] * adj2[None, :, :, :], axis=1)

    x2 = jax.lax.dot_general(
        w2f_ref[...], xt2.reshape(64, NB), (((1,), (0,)), ((), ())),
        preferred_element_type=jnp.float32)
    x2 = x2 * g2_ref[...] + bb2_ref[...]
    x2 = x2 / (1.0 + jnp.abs(x2))

    out_ref[...] = jax.lax.dot_general(
        wc_ref[...], x2, (((1,), (0,)), ((), ())),
        preferred_element_type=jnp.float32) + bc_ref[...]


@jax.jit
def kernel(x, neighbor, W1, b1, gamma1, beta1, W2, b2, gamma2, beta2, Wc, bc):
    N, _, F = x.shape
    D = neighbor.shape[2]
    xb = x.reshape(N, F)
    nb = neighbor.reshape(N, D, F)

    inv = 1.0 / jnp.sqrt(1.0 + 1e-5)
    w1t = W1.reshape(64, F).T
    g1 = jnp.repeat(gamma1, 16)
    g1v = (g1 * inv)[None, :]
    b1v = (b1 * inv * g1 + jnp.repeat(beta1, 16))[None, :]

    w2f = W2.reshape(32, 64)
    g2 = (gamma2 * inv)[:, None]
    bb2 = (b2 * inv * gamma2 + beta2)[:, None]
    bcr = bc[:, None]

    grid = (pl.cdiv(N, _NB),)
    out_t = pl.pallas_call(
        _ktrans_block,
        grid=grid,
        in_specs=[
            pl.BlockSpec((_NB, F), lambda i: (i, 0)),
            pl.BlockSpec((_NB, D, F), lambda i: (i, 0, 0)),
            pl.BlockSpec((F, 64), lambda i: (0, 0)),
            pl.BlockSpec((1, 64), lambda i: (0, 0)),
            pl.BlockSpec((1, 64), lambda i: (0, 0)),
            pl.BlockSpec((32, 64), lambda i: (0, 0)),
            pl.BlockSpec((32, 1), lambda i: (0, 0)),
            pl.BlockSpec((32, 1), lambda i: (0, 0)),
            pl.BlockSpec((10, 32), lambda i: (0, 0)),
            pl.BlockSpec((10, 1), lambda i: (0, 0)),
        ],
        out_specs=pl.BlockSpec((10, _NB), lambda i: (0, i)),
        out_shape=jax.ShapeDtypeStruct((10, N), jnp.float32),
    )(xb, nb, w1t, g1v, b1v, w2f, g2, bb2, Wc, bcr)
    return out_t.T
